# baseline (speedup 1.0000x reference)
import jax
import jax.numpy as jnp
from jax.experimental import pallas as pl
from jax.experimental.pallas import tpu as pltpu

_CHUNKS = ((0, 504), (504, 1000), (1504, 2000), (3504, 3000), (6504, 3496))
_MAXC = 3496


def _body(x_hbm, w_ref, b_ref, o_hbm, xb0, xb1, ob0, ob1, rsem, wsem):
    xbufs = (xb0, xb1)
    obufs = (ob0, ob1)

    def rd(i):
        off, sz = _CHUNKS[i]
        return pltpu.make_async_copy(
            x_hbm.at[pl.ds(off, sz), :],
            xbufs[i % 2].at[pl.ds(0, sz), :],
            rsem.at[i % 2],
        )

    def wr(i):
        off, sz = _CHUNKS[i]
        return pltpu.make_async_copy(
            obufs[i % 2].at[pl.ds(0, sz), :],
            o_hbm.at[pl.ds(off, sz), :],
            wsem.at[i % 2],
        )

    n = len(_CHUNKS)
    rd(0).start()
    rd(1).start()
    for i in range(n):
        off, sz = _CHUNKS[i]
        rd(i).wait()
        if i >= 2:
            wr(i - 2).wait()
        acc = jnp.dot(
            xbufs[i % 2][pl.ds(0, sz), :],
            w_ref[...],
            preferred_element_type=jnp.float32,
        )
        obufs[i % 2][pl.ds(0, sz), :] = jnp.maximum(acc + b_ref[...], 0.0)
        wr(i).start()
        if i + 2 < n:
            rd(i + 2).start()
    wr(n - 2).wait()
    wr(n - 1).wait()


def kernel(node_features, edge_index, edge_features, W, b):
    del edge_index, edge_features
    n, k = node_features.shape
    d = W.shape[1]
    b2 = b.reshape(1, d)
    return pl.pallas_call(
        _body,
        in_specs=[
            pl.BlockSpec(memory_space=pltpu.MemorySpace.HBM),
            pl.BlockSpec(memory_space=pltpu.MemorySpace.VMEM),
            pl.BlockSpec(memory_space=pltpu.MemorySpace.VMEM),
        ],
        out_specs=pl.BlockSpec(memory_space=pltpu.MemorySpace.HBM),
        out_shape=jax.ShapeDtypeStruct((n, d), jnp.float32),
        scratch_shapes=[
            pltpu.VMEM((_MAXC, k), jnp.float32),
            pltpu.VMEM((_MAXC, k), jnp.float32),
            pltpu.VMEM((_MAXC, d), jnp.float32),
            pltpu.VMEM((_MAXC, d), jnp.float32),
            pltpu.SemaphoreType.DMA((2,)),
            pltpu.SemaphoreType.DMA((2,)),
        ],
    )(node_features, W, b2)


# speedup vs baseline: 1.1172x; 1.1172x over previous
import jax
import jax.numpy as jnp
from jax.experimental import pallas as pl
from jax.experimental.pallas import tpu as pltpu


def _fused_fc_relu(x_ref, w_ref, b_ref, o_ref):
    acc = jnp.dot(x_ref[...], w_ref[...], preferred_element_type=jnp.float32)
    o_ref[...] = jnp.maximum(acc + b_ref[...], 0.0)


def kernel(node_features, edge_index, edge_features, W, b):
    del edge_index, edge_features
    n, k = node_features.shape
    d = W.shape[1]
    block = 4000
    grid = pl.cdiv(n, block)
    b2 = b.reshape(1, d)
    return pl.pallas_call(
        _fused_fc_relu,
        grid=(grid,),
        in_specs=[
            pl.BlockSpec((block, k), lambda i: (i, 0)),
            pl.BlockSpec((k, d), lambda i: (0, 0)),
            pl.BlockSpec((1, d), lambda i: (0, 0)),
        ],
        out_specs=pl.BlockSpec((block, d), lambda i: (i, 0)),
        out_shape=jax.ShapeDtypeStruct((n, d), jnp.float32),
        compiler_params=pltpu.CompilerParams(
            dimension_semantics=("parallel",)
        ),
    )(node_features, W, b2)
